# baseline (speedup 1.0000x reference)
import functools

import jax
import jax.numpy as jnp
from jax import lax
from jax.scipy.linalg import block_diag
from jax.experimental import pallas as pl
from jax.experimental.pallas import tpu as pltpu
from jax.experimental.pallas import tpu_sc as plsc

_N = 10000
_E = 320000
_D_IN = 128
_D = 32

_NC, _NS = 2, 16
_NW = _NC * _NS
_CHUNK = 128
_CPW = 80
_EP = _NW * _CPW * _CHUNK
_NP = 10240
_RPT = _NP // _NS
_PR = _NP // 4
_CR = _NP // 128

_PBLK = _PR // 8
_NBLK = 8

_GRP = 1
_EPD = _GRP * _CHUNK
_CPD = _CPW // _GRP
_NBUF = 10
_LOOKA = 8


@functools.lru_cache(maxsize=None)
def _make_sc_segsum(with_count: bool):
    mesh = plsc.VectorSubcoreMesh(core_axis_name="c", subcore_axis_name="s")
    out_type = [jax.ShapeDtypeStruct((_NC * _NP, _D), jnp.float32)]
    scratch = [
        pltpu.VMEM((_CPD, _EPD), jnp.int32),
        pltpu.VMEM((_CPD, _EPD), jnp.int32),
        pltpu.VMEM((_NBUF, _EPD, _D), jnp.float32),
        pltpu.VMEM_SHARED((_NP, _D), jnp.float32),
        pltpu.VMEM_SHARED((_NP, _D), jnp.float32),
        [pltpu.SemaphoreType.DMA] * _NBUF,
        [pltpu.SemaphoreType.DMA] * _NBUF,
    ]
    if with_count:
        out_type.append(jax.ShapeDtypeStruct((_NC * _NP, _D), jnp.float32))
        scratch += [
            pltpu.VMEM((_EPD, _D), jnp.float32),
            pltpu.VMEM_SHARED((_NP, _D), jnp.float32),
        ]

    def body(y_hbm, src_hbm, dst_hbm, *rest):
        if with_count:
            (agg_out, cnt_out, src_v, dst_v, rows_v, agg_sh, y_sh,
             gsem, ssem, one_v, cnt_sh) = rest
        else:
            (agg_out, src_v, dst_v, rows_v, agg_sh, y_sh,
             gsem, ssem) = rest

        c = lax.axis_index("c")
        s = lax.axis_index("s")
        w = s * _NC + c

        pltpu.sync_copy(src_hbm.at[pl.ds(w * _CPD, _CPD)], src_v)
        pltpu.sync_copy(dst_hbm.at[pl.ds(w * _CPD, _CPD)], dst_v)
        pltpu.sync_copy(y_hbm.at[pl.ds(s * _RPT, _RPT)],
                        y_sh.at[pl.ds(s * _RPT, _RPT)])

        zv = jnp.zeros((16,), jnp.float32)

        def fill_rows(i, _):
            rows_v[0, i, pl.ds(0, 16)] = zv
            rows_v[0, i, pl.ds(16, 16)] = zv
            return 0
        lax.fori_loop(0, _CHUNK, fill_rows, 0)

        if with_count:
            ov = jnp.ones((16,), jnp.float32)

            def fill_ones(i, _):
                one_v[i, pl.ds(0, 16)] = ov
                one_v[i, pl.ds(16, 16)] = ov
                return 0
            lax.fori_loop(0, _EPD, fill_ones, 0)

        for k in range(_RPT // _CHUNK):
            base = s * _RPT + k * _CHUNK
            pltpu.sync_copy(rows_v.at[0], agg_sh.at[pl.ds(base, _CHUNK)])
            if with_count:
                pltpu.sync_copy(rows_v.at[0], cnt_sh.at[pl.ds(base, _CHUNK)])
        plsc.subcore_barrier()

        def gather(j, b):
            pltpu.async_copy(y_sh.at[src_v.at[j]], rows_v.at[b], gsem[b])

        def wait_gather(b):
            pltpu.make_async_copy(
                y_sh.at[src_v.at[0]], rows_v.at[b], gsem[b]).wait()

        def scatter(j, b):
            pltpu.async_copy(rows_v.at[b], agg_sh.at[dst_v.at[j]], ssem[b],
                             add=True)
            if with_count:
                pltpu.async_copy(one_v, cnt_sh.at[dst_v.at[j]], ssem[b],
                                 add=True)

        def wait_scatter(b):
            pltpu.make_async_copy(
                rows_v.at[b], agg_sh.at[dst_v.at[0]], ssem[b]).wait()
            if with_count:
                pltpu.make_async_copy(
                    one_v, cnt_sh.at[dst_v.at[0]], ssem[b]).wait()

        lag = _NBUF - _LOOKA
        npb = _CPD // _NBUF

        for b in range(_LOOKA):
            gather(b, b)

        def eloop(i, _):
            for b in range(_NBUF):
                j = i * _NBUF + b
                bf = (b + _LOOKA) % _NBUF
                if b >= lag:
                    wait_scatter(bf)
                else:
                    @pl.when(i > 0)
                    def _(bf=bf):
                        wait_scatter(bf)
                if b < (_CPD - _LOOKA) % _NBUF:
                    gather(j + _LOOKA, bf)
                else:
                    @pl.when(i + 1 < npb)
                    def _(j=j, bf=bf):
                        gather(j + _LOOKA, bf)
                wait_gather(b)
                scatter(j, b)
            return 0
        lax.fori_loop(0, npb, eloop, 0)
        for b in range(lag):
            wait_scatter((_CPD - lag + b) % _NBUF)
        plsc.subcore_barrier()

        pltpu.sync_copy(agg_sh.at[pl.ds(s * _RPT, _RPT)],
                        agg_out.at[pl.ds(c * _NP + s * _RPT, _RPT)])
        if with_count:
            pltpu.sync_copy(cnt_sh.at[pl.ds(s * _RPT, _RPT)],
                            cnt_out.at[pl.ds(c * _NP + s * _RPT, _RPT)])

    return pl.kernel(
        body, out_type=out_type, mesh=mesh, scratch_types=scratch,
        compiler_params=pltpu.CompilerParams(use_tc_tiling_on_sc=False))


def _tc1_body(x_ref, wl_ref, wr_ref, b_ref, y_ref, r_ref):
    xb = x_ref[...]
    y_ref[...] = jnp.dot(xb, wl_ref[...], preferred_element_type=jnp.float32)
    r_ref[...] = jnp.dot(xb, wr_ref[...],
                         preferred_element_type=jnp.float32) + b_ref[...]


def _inv_packed(cntp_ref):
    return 1.0 / jnp.maximum(cntp_ref[0] + cntp_ref[1], 1.0)


def _tc2_body(aggp_ref, cntp_ref, r1_ref, wl_ref, wr_ref, b_ref,
              y2_ref, r2_ref):
    agg = aggp_ref[0] + aggp_ref[1]
    h = jnp.maximum(agg * _inv_packed(cntp_ref) + r1_ref[...], 0.0)
    y2_ref[...] = jnp.dot(h, wl_ref[...], preferred_element_type=jnp.float32)
    r2_ref[...] = jnp.dot(h, wr_ref[...],
                          preferred_element_type=jnp.float32) + b_ref[...]


def _tc3_body(aggp_ref, cntp_ref, r2_ref, w_ref, b_ref, o_ref):
    agg = aggp_ref[0] + aggp_ref[1]
    h2 = agg * _inv_packed(cntp_ref) + r2_ref[...]
    o_ref[...] = jnp.dot(h2, w_ref[...],
                         preferred_element_type=jnp.float32) + b_ref[...]


def _rows_spec(width):
    return pl.BlockSpec((_PBLK, width), lambda i: (i, 0))


def _part_spec(rows, width):
    return pl.BlockSpec((2, rows, width), lambda i: (0, i, 0))


def _full_spec(shape):
    return pl.BlockSpec(shape, lambda i: tuple(0 for _ in shape))


_CBLK = _CR // _NBLK

_tc1 = pl.pallas_call(
    _tc1_body,
    grid=(_NBLK,),
    in_specs=[pl.BlockSpec((_PBLK, 4 * _D_IN), lambda i: (i, 0)),
              _full_spec((4 * _D_IN, 128)), _full_spec((4 * _D_IN, 128)),
              _full_spec((1, 128))],
    out_specs=[_rows_spec(128), _rows_spec(128)],
    out_shape=[jax.ShapeDtypeStruct((_PR, 128), jnp.float32)] * 2,
)

_tc2 = pl.pallas_call(
    _tc2_body,
    grid=(_NBLK,),
    in_specs=[_part_spec(_PBLK, 128), _part_spec(_PBLK, 128),
              _rows_spec(128), _full_spec((128, 128)), _full_spec((128, 128)),
              _full_spec((1, 128))],
    out_specs=[_rows_spec(128), _rows_spec(128)],
    out_shape=[jax.ShapeDtypeStruct((_PR, 128), jnp.float32)] * 2,
)

_tc3 = pl.pallas_call(
    _tc3_body,
    grid=(_NBLK,),
    in_specs=[_part_spec(_PBLK, 128), _part_spec(_PBLK, 128),
              _rows_spec(128), _full_spec((128, 4)), _full_spec((1, 1))],
    out_specs=_rows_spec(4),
    out_shape=jax.ShapeDtypeStruct((_PR, 4), jnp.float32),
)


def kernel(x, edge_index, W1_l, W1_r, b1, W2_l, W2_r, b2, W3, b3):
    src = edge_index[0].astype(jnp.int32)
    dst = edge_index[1].astype(jnp.int32)
    pad = _EP - _E
    src2d = jnp.concatenate([src, jnp.zeros((pad,), jnp.int32)]
                            ).reshape(_EP // _EPD, _EPD)
    dst2d = jnp.concatenate([dst, jnp.full((pad,), _N, jnp.int32)]
                            ).reshape(_EP // _EPD, _EPD)
    xp = jnp.pad(x, ((0, _NP - _N), (0, 0))).reshape(_PR, 4 * _D_IN)

    W1lB = block_diag(W1_l, W1_l, W1_l, W1_l)
    W1rB = block_diag(W1_r, W1_r, W1_r, W1_r)
    W2lB = block_diag(W2_l, W2_l, W2_l, W2_l)
    W2rB = block_diag(W2_r, W2_r, W2_r, W2_r)
    W3B = block_diag(W3, W3, W3, W3)
    b1t = jnp.tile(b1, 4).reshape(1, 128)
    b2t = jnp.tile(b2, 4).reshape(1, 128)

    y1p, r1p = _tc1(xp, W1lB, W1rB, b1t)
    agg1, cnt = _make_sc_segsum(True)(y1p.reshape(_NP, _D), src2d, dst2d)
    aggp1 = agg1.reshape(_NC, _PR, 128)
    cntp = cnt.reshape(_NC, _PR, 128)
    y2p, r2p = _tc2(aggp1, cntp, r1p, W2lB, W2rB, b2t)
    (agg2,) = jax.tree.leaves(
        _make_sc_segsum(False)(y2p.reshape(_NP, _D), src2d, dst2d))
    outp = _tc3(agg2.reshape(_NC, _PR, 128), cntp, r2p, W3B,
                b3.reshape(1, 1))
    return outp.reshape(_NP, 1)[:_N]


# speedup vs baseline: 1.0340x; 1.0340x over previous
import functools

import jax
import jax.numpy as jnp
from jax import lax
from jax.scipy.linalg import block_diag
from jax.experimental import pallas as pl
from jax.experimental.pallas import tpu as pltpu
from jax.experimental.pallas import tpu_sc as plsc

_N = 10000
_E = 320000
_D_IN = 128
_D = 32

_NC, _NS = 2, 16
_NW = _NC * _NS
_CHUNK = 128
_CPW = 80
_EP = _NW * _CPW * _CHUNK
_NP = 10240
_RPT = _NP // _NS
_PR = _NP // 4
_CR = _NP // 128

_PBLK = _PR // 8
_NBLK = 8

_GRP = 1
_EPD = _GRP * _CHUNK
_CPD = _CPW // _GRP
_NBUF = 10
_LOOKA = 7


@functools.lru_cache(maxsize=None)
def _make_sc_segsum(with_count: bool):
    mesh = plsc.VectorSubcoreMesh(core_axis_name="c", subcore_axis_name="s")
    out_type = [jax.ShapeDtypeStruct((_NC * _NP, _D), jnp.float32)]
    scratch = [
        pltpu.VMEM((_CPD, _EPD), jnp.int32),
        pltpu.VMEM((_CPD, _EPD), jnp.int32),
        pltpu.VMEM((_NBUF, _EPD, _D), jnp.float32),
        pltpu.VMEM_SHARED((_NP, _D), jnp.float32),
        pltpu.VMEM_SHARED((_NP, _D), jnp.float32),
        [pltpu.SemaphoreType.DMA] * _NBUF,
        [pltpu.SemaphoreType.DMA] * _NBUF,
    ]
    if with_count:
        out_type.append(jax.ShapeDtypeStruct((_NC * _NP, _D), jnp.float32))
        scratch += [
            pltpu.VMEM((_EPD, _D), jnp.float32),
            pltpu.VMEM_SHARED((_NP, _D), jnp.float32),
        ]

    def body(y_hbm, src_hbm, dst_hbm, *rest):
        if with_count:
            (agg_out, cnt_out, src_v, dst_v, rows_v, agg_sh, y_sh,
             gsem, ssem, one_v, cnt_sh) = rest
        else:
            (agg_out, src_v, dst_v, rows_v, agg_sh, y_sh,
             gsem, ssem) = rest

        c = lax.axis_index("c")
        s = lax.axis_index("s")
        w = s * _NC + c

        pltpu.async_copy(src_hbm.at[pl.ds(w * _CPD, _CPD)], src_v, gsem[0])
        pltpu.async_copy(dst_hbm.at[pl.ds(w * _CPD, _CPD)], dst_v, gsem[1])
        pltpu.async_copy(y_hbm.at[pl.ds(s * _RPT, _RPT)],
                         y_sh.at[pl.ds(s * _RPT, _RPT)], gsem[2])

        zv = jnp.zeros((16,), jnp.float32)

        def fill_rows(i, _):
            rows_v[0, i, pl.ds(0, 16)] = zv
            rows_v[0, i, pl.ds(16, 16)] = zv
            return 0
        lax.fori_loop(0, _CHUNK, fill_rows, 0)

        if with_count:
            ov = jnp.ones((16,), jnp.float32)

            def fill_ones(i, _):
                one_v[i, pl.ds(0, 16)] = ov
                one_v[i, pl.ds(16, 16)] = ov
                return 0
            lax.fori_loop(0, _EPD, fill_ones, 0)

        pltpu.make_async_copy(
            src_hbm.at[pl.ds(0, _CPD)], src_v, gsem[0]).wait()
        pltpu.make_async_copy(
            dst_hbm.at[pl.ds(0, _CPD)], dst_v, gsem[1]).wait()
        pltpu.make_async_copy(
            y_hbm.at[pl.ds(0, _RPT)], y_sh.at[pl.ds(0, _RPT)],
            gsem[2]).wait()
        nz = _RPT // _CHUNK
        for k in range(nz):
            base = s * _RPT + k * _CHUNK
            pltpu.async_copy(rows_v.at[0], agg_sh.at[pl.ds(base, _CHUNK)],
                             ssem[k])
            if with_count:
                pltpu.async_copy(rows_v.at[0], cnt_sh.at[pl.ds(base, _CHUNK)],
                                 ssem[nz + k])
        for k in range(nz):
            pltpu.make_async_copy(
                rows_v.at[0], agg_sh.at[pl.ds(0, _CHUNK)], ssem[k]).wait()
            if with_count:
                pltpu.make_async_copy(
                    rows_v.at[0], cnt_sh.at[pl.ds(0, _CHUNK)],
                    ssem[nz + k]).wait()
        plsc.subcore_barrier()

        def gather(j, b):
            pltpu.async_copy(y_sh.at[src_v.at[j]], rows_v.at[b], gsem[b])

        def wait_gather(b):
            pltpu.make_async_copy(
                y_sh.at[src_v.at[0]], rows_v.at[b], gsem[b]).wait()

        def scatter(j, b):
            pltpu.async_copy(rows_v.at[b], agg_sh.at[dst_v.at[j]], ssem[b],
                             add=True)
            if with_count:
                pltpu.async_copy(one_v, cnt_sh.at[dst_v.at[j]], ssem[b],
                                 add=True)

        def wait_scatter(b):
            pltpu.make_async_copy(
                rows_v.at[b], agg_sh.at[dst_v.at[0]], ssem[b]).wait()
            if with_count:
                pltpu.make_async_copy(
                    one_v, cnt_sh.at[dst_v.at[0]], ssem[b]).wait()

        lag = _NBUF - _LOOKA
        npb = _CPD // _NBUF

        for b in range(_LOOKA):
            gather(b, b)

        def eloop(i, _):
            for b in range(_NBUF):
                j = i * _NBUF + b
                bf = (b + _LOOKA) % _NBUF
                if b >= lag:
                    wait_scatter(bf)
                else:
                    @pl.when(i > 0)
                    def _(bf=bf):
                        wait_scatter(bf)
                if b < (_CPD - _LOOKA) % _NBUF:
                    gather(j + _LOOKA, bf)
                else:
                    @pl.when(i + 1 < npb)
                    def _(j=j, bf=bf):
                        gather(j + _LOOKA, bf)
                wait_gather(b)
                scatter(j, b)
            return 0
        lax.fori_loop(0, npb, eloop, 0)
        for b in range(lag):
            wait_scatter((_CPD - lag + b) % _NBUF)
        plsc.subcore_barrier()

        pltpu.sync_copy(agg_sh.at[pl.ds(s * _RPT, _RPT)],
                        agg_out.at[pl.ds(c * _NP + s * _RPT, _RPT)])
        if with_count:
            pltpu.sync_copy(cnt_sh.at[pl.ds(s * _RPT, _RPT)],
                            cnt_out.at[pl.ds(c * _NP + s * _RPT, _RPT)])

    return pl.kernel(
        body, out_type=out_type, mesh=mesh, scratch_types=scratch,
        compiler_params=pltpu.CompilerParams(use_tc_tiling_on_sc=False))


def _tc1_body(x_ref, wl_ref, wr_ref, b_ref, y_ref, r_ref):
    xb = x_ref[...]
    y_ref[...] = jnp.dot(xb, wl_ref[...], preferred_element_type=jnp.float32)
    r_ref[...] = jnp.dot(xb, wr_ref[...],
                         preferred_element_type=jnp.float32) + b_ref[...]


def _inv_packed(cntp_ref):
    return 1.0 / jnp.maximum(cntp_ref[0] + cntp_ref[1], 1.0)


def _tc2_body(aggp_ref, cntp_ref, r1_ref, wl_ref, wr_ref, b_ref,
              y2_ref, r2_ref):
    agg = aggp_ref[0] + aggp_ref[1]
    h = jnp.maximum(agg * _inv_packed(cntp_ref) + r1_ref[...], 0.0)
    y2_ref[...] = jnp.dot(h, wl_ref[...], preferred_element_type=jnp.float32)
    r2_ref[...] = jnp.dot(h, wr_ref[...],
                          preferred_element_type=jnp.float32) + b_ref[...]


def _tc3_body(aggp_ref, cntp_ref, r2_ref, w_ref, b_ref, o_ref):
    agg = aggp_ref[0] + aggp_ref[1]
    h2 = agg * _inv_packed(cntp_ref) + r2_ref[...]
    o_ref[...] = jnp.dot(h2, w_ref[...],
                         preferred_element_type=jnp.float32) + b_ref[...]


def _rows_spec(width):
    return pl.BlockSpec((_PBLK, width), lambda i: (i, 0))


def _part_spec(rows, width):
    return pl.BlockSpec((2, rows, width), lambda i: (0, i, 0))


def _full_spec(shape):
    return pl.BlockSpec(shape, lambda i: tuple(0 for _ in shape))


_CBLK = _CR // _NBLK

_tc1 = pl.pallas_call(
    _tc1_body,
    grid=(_NBLK,),
    in_specs=[pl.BlockSpec((_PBLK, 4 * _D_IN), lambda i: (i, 0)),
              _full_spec((4 * _D_IN, 128)), _full_spec((4 * _D_IN, 128)),
              _full_spec((1, 128))],
    out_specs=[_rows_spec(128), _rows_spec(128)],
    out_shape=[jax.ShapeDtypeStruct((_PR, 128), jnp.float32)] * 2,
)

_tc2 = pl.pallas_call(
    _tc2_body,
    grid=(_NBLK,),
    in_specs=[_part_spec(_PBLK, 128), _part_spec(_PBLK, 128),
              _rows_spec(128), _full_spec((128, 128)), _full_spec((128, 128)),
              _full_spec((1, 128))],
    out_specs=[_rows_spec(128), _rows_spec(128)],
    out_shape=[jax.ShapeDtypeStruct((_PR, 128), jnp.float32)] * 2,
)

_tc3 = pl.pallas_call(
    _tc3_body,
    grid=(_NBLK,),
    in_specs=[_part_spec(_PBLK, 128), _part_spec(_PBLK, 128),
              _rows_spec(128), _full_spec((128, 4)), _full_spec((1, 1))],
    out_specs=_rows_spec(4),
    out_shape=jax.ShapeDtypeStruct((_PR, 4), jnp.float32),
)


def kernel(x, edge_index, W1_l, W1_r, b1, W2_l, W2_r, b2, W3, b3):
    src = edge_index[0].astype(jnp.int32)
    dst = edge_index[1].astype(jnp.int32)
    pad = _EP - _E
    src2d = jnp.concatenate([src, jnp.zeros((pad,), jnp.int32)]
                            ).reshape(_EP // _EPD, _EPD)
    dst2d = jnp.concatenate([dst, jnp.full((pad,), _N, jnp.int32)]
                            ).reshape(_EP // _EPD, _EPD)
    xp = jnp.pad(x, ((0, _NP - _N), (0, 0))).reshape(_PR, 4 * _D_IN)

    W1lB = block_diag(W1_l, W1_l, W1_l, W1_l)
    W1rB = block_diag(W1_r, W1_r, W1_r, W1_r)
    W2lB = block_diag(W2_l, W2_l, W2_l, W2_l)
    W2rB = block_diag(W2_r, W2_r, W2_r, W2_r)
    W3B = block_diag(W3, W3, W3, W3)
    b1t = jnp.tile(b1, 4).reshape(1, 128)
    b2t = jnp.tile(b2, 4).reshape(1, 128)

    y1p, r1p = _tc1(xp, W1lB, W1rB, b1t)
    agg1, cnt = _make_sc_segsum(True)(y1p.reshape(_NP, _D), src2d, dst2d)
    aggp1 = agg1.reshape(_NC, _PR, 128)
    cntp = cnt.reshape(_NC, _PR, 128)
    y2p, r2p = _tc2(aggp1, cntp, r1p, W2lB, W2rB, b2t)
    (agg2,) = jax.tree.leaves(
        _make_sc_segsum(False)(y2p.reshape(_NP, _D), src2d, dst2d))
    outp = _tc3(agg2.reshape(_NC, _PR, 128), cntp, r2p, W3B,
                b3.reshape(1, 1))
    return outp.reshape(_NP, 1)[:_N]
